# baseline (speedup 1.0000x reference)
import functools

import jax
import jax.numpy as jnp
from jax import lax
from jax.experimental import pallas as pl
from jax.experimental.pallas import tpu as pltpu
from jax.experimental.pallas import tpu_sc as plsc

B_USERS = 4096
N_ITEMS = 4096

NC = 2
NS = 16
NW = NC * NS
LANES = 16

SB_ROWS = 16
NSB = B_USERS // SB_ROWS
CHUNK = 2048
STARTS_PAD = 272

SPLIT = 2
H_ROWS = B_USERS // SPLIT
H_NSB = NSB // SPLIT
H_SB_PER_W = H_NSB // NW

_MESH = plsc.VectorSubcoreMesh(core_axis_name="c", subcore_axis_name="s")


def _make_densify(row0):
    @functools.partial(
        pl.kernel,
        out_type=jax.ShapeDtypeStruct((H_ROWS, N_ITEMS // 2), jnp.float32),
        mesh=_MESH,
        scratch_types=[
            pltpu.VMEM((CHUNK,), jnp.int32),
            pltpu.VMEM((CHUNK,), jnp.int32),
            pltpu.VMEM((CHUNK,), jnp.float32),
            pltpu.VMEM((STARTS_PAD,), jnp.int32),
            pltpu.VMEM((SB_ROWS, N_ITEMS), jnp.float32),
            pltpu.VMEM((SB_ROWS, N_ITEMS // 2), jnp.float32),
            pltpu.SemaphoreType.DMA,
        ],
        compiler_params=pltpu.CompilerParams(needs_layout_passes=False),
    )
    def _densify_sc(rows_hbm, cols_hbm, vals_hbm, starts_hbm, x_hbm,
                    rows_v, cols_v, vals_v, starts_v, xbuf, xbuf16, sem):
        wid = lax.axis_index("s") * NC + lax.axis_index("c")

        pltpu.sync_copy(starts_hbm, starts_v)

        zv = jnp.zeros((LANES,), jnp.float32)

        def _memset_row(r):
            def _body(i, carry):
                for u in range(8):
                    xbuf[r, pl.ds((i * 8 + u) * LANES, LANES)] = zv
                return carry
            lax.fori_loop(0, N_ITEMS // (8 * LANES), _body, 0)

        for r in range(SB_ROWS):
            _memset_row(r)

        for t in range(H_SB_PER_W):
            lbase = (wid * H_SB_PER_W + t) * SB_ROWS
            sb = row0 // SB_ROWS + wid * H_SB_PER_W + t
            base = row0 + lbase
            svec = starts_v[pl.ds(sb, LANES)]
            s0 = svec[0]
            s1 = svec[1]
            c0 = (s0 // 8) * 8
            nch = (s1 - c0 + CHUNK - 1) // CHUNK

            def _stage(off):
                d1 = pltpu.async_copy(
                    rows_hbm.at[pl.ds(off, CHUNK)], rows_v, sem)
                d2 = pltpu.async_copy(
                    cols_hbm.at[pl.ds(off, CHUNK)], cols_v, sem)
                d3 = pltpu.async_copy(
                    vals_hbm.at[pl.ds(off, CHUNK)], vals_v, sem)
                d1.wait()
                d2.wait()
                d3.wait()

            def _masked_idx(i):
                rv = rows_v[pl.ds(i * LANES, LANES)]
                cv = cols_v[pl.ds(i * LANES, LANES)]
                m = (rv >= base) & (rv < base + SB_ROWS)
                ri = jnp.where(m, rv - base, 0)
                cp = ((cv & ~31) | ((cv >> 1) & 15)) | ((cv & 1) << 4)
                ci = jnp.where(m, cp, 0)
                return m, ri, ci

            def _chunk_add(j, carry):
                _stage(c0 + j * CHUNK)

                def _vec(i, inner):
                    m, ri, ci = _masked_idx(i)
                    vv = vals_v[pl.ds(i * LANES, LANES)]
                    plsc.addupdate_scatter(
                        xbuf, [ri, ci], jnp.where(m, vv, 0.0), mask=m)
                    return inner

                lax.fori_loop(0, CHUNK // LANES, _vec, 0)
                return carry

            lax.fori_loop(0, nch, _chunk_add, 0)

            for r in range(SB_ROWS):
                def _pack_row(g, carry, _r=r):
                    for u in range(2):
                        p = (g * 2 + u) * (2 * LANES)
                        a = xbuf[_r, pl.ds(p, LANES)]
                        b = xbuf[_r, pl.ds(p + LANES, LANES)]
                        pk = plsc.pack(
                            a, b, format=plsc.PackFormat.INTERLEAVED)
                        xbuf16[_r, pl.ds(p // 2, LANES)] = plsc.bitcast(
                            pk, jnp.float32)
                    return carry
                lax.fori_loop(0, N_ITEMS // (4 * LANES), _pack_row, 0)

            pltpu.sync_copy(xbuf16, x_hbm.at[pl.ds(lbase, SB_ROWS)])

            def _chunk_zero(j, carry):
                @pl.when(nch > 1)
                def _():
                    _stage(c0 + j * CHUNK)

                def _vec(i, inner):
                    m, ri, ci = _masked_idx(i)
                    plsc.store_scatter(xbuf, [ri, ci], zv, mask=m)
                    return inner

                lax.fori_loop(0, CHUNK // LANES, _vec, 0)
                return carry

            lax.fori_loop(0, nch, _chunk_zero, 0)

    return _densify_sc


_densify_halves = tuple(_make_densify(h * H_ROWS) for h in range(SPLIT))


_BM = 512


def _matmul_first(x, w16):
    k = x.shape[1]
    n = w16.shape[1]

    def body(x_ref, w_ref, o_ref):
        o_ref[...] = jnp.dot(
            x_ref[...], w_ref[...], preferred_element_type=jnp.float32)

    return pl.pallas_call(
        body,
        grid=(H_ROWS // _BM,),
        in_specs=[
            pl.BlockSpec((_BM, k), lambda i: (i, 0)),
            pl.BlockSpec((k, n), lambda i: (0, 0)),
        ],
        out_specs=pl.BlockSpec((_BM, n), lambda i: (i, 0)),
        out_shape=jax.ShapeDtypeStruct((B_USERS, N_ITEMS), jnp.float32),
        compiler_params=pltpu.CompilerParams(
            dimension_semantics=("arbitrary",),
            vmem_limit_bytes=100 * 1024 * 1024),
    )(x, w16)


def _matmul_rest(ybuf, x, w16, h):
    k = x.shape[1]
    n = w16.shape[1]
    nblk = H_ROWS // _BM

    def body(y_ref, x_ref, w_ref, o_ref):
        del y_ref
        o_ref[...] = jnp.dot(
            x_ref[...], w_ref[...], preferred_element_type=jnp.float32)

    return pl.pallas_call(
        body,
        grid=(nblk,),
        in_specs=[
            pl.BlockSpec(memory_space=pltpu.MemorySpace.HBM),
            pl.BlockSpec((_BM, k), lambda i: (i, 0)),
            pl.BlockSpec((k, n), lambda i: (0, 0)),
        ],
        out_specs=pl.BlockSpec((_BM, n), lambda i, _h=h: (i + _h * nblk, 0)),
        out_shape=jax.ShapeDtypeStruct((B_USERS, N_ITEMS), jnp.float32),
        input_output_aliases={0: 0},
        compiler_params=pltpu.CompilerParams(
            dimension_semantics=("arbitrary",),
            vmem_limit_bytes=100 * 1024 * 1024),
    )(ybuf, x, w16)


def kernel(vals, W, rows, cols):
    rows32 = rows.astype(jnp.int32)
    cols32 = cols.astype(jnp.int32)
    vals32 = vals.astype(jnp.float32)
    nnz = rows32.shape[0]

    bounds = jnp.arange(NSB + 1, dtype=jnp.int32) * SB_ROWS
    starts = jnp.sum(rows32[None, :] < bounds[:, None], axis=1,
                     dtype=jnp.int32)
    starts_p = jnp.concatenate(
        [starts, jnp.full((STARTS_PAD - NSB - 1,), nnz, jnp.int32)])

    rows_p = jnp.concatenate(
        [rows32, jnp.full((CHUNK,), B_USERS, jnp.int32)])
    cols_p = jnp.concatenate([cols32, jnp.zeros((CHUNK,), jnp.int32)])
    vals_p = jnp.concatenate([vals32, jnp.zeros((CHUNK,), jnp.float32)])

    w16 = W.astype(jnp.bfloat16)
    xs = [
        lax.bitcast_convert_type(
            dh(rows_p, cols_p, vals_p, starts_p), jnp.bfloat16
        ).reshape(H_ROWS, N_ITEMS)
        for dh in _densify_halves
    ]
    y = _matmul_first(xs[0], w16)
    for h in range(1, SPLIT):
        y = _matmul_rest(y, xs[h], w16, h)
    return y


# speedup vs baseline: 2.4087x; 2.4087x over previous
import functools

import jax
import jax.numpy as jnp
from jax import lax
from jax.experimental import pallas as pl
from jax.experimental.pallas import tpu as pltpu
from jax.experimental.pallas import tpu_sc as plsc

B_USERS = 4096
N_ITEMS = 4096

NC = 2
NS = 16
NW = NC * NS
LANES = 16

SB_ROWS = 16
NSB = B_USERS // SB_ROWS
CHUNK = 2048
STARTS_PAD = 272

SPLIT = 2
H_ROWS = B_USERS // SPLIT
H_NSB = NSB // SPLIT
H_SB_PER_W = H_NSB // NW

_MESH = plsc.VectorSubcoreMesh(core_axis_name="c", subcore_axis_name="s")


def _make_densify(row0):
    @functools.partial(
        pl.kernel,
        out_type=jax.ShapeDtypeStruct((H_ROWS, N_ITEMS), jnp.float32),
        mesh=_MESH,
        scratch_types=[
            pltpu.VMEM((CHUNK,), jnp.int32),
            pltpu.VMEM((CHUNK,), jnp.int32),
            pltpu.VMEM((CHUNK,), jnp.float32),
            pltpu.VMEM((STARTS_PAD,), jnp.int32),
            pltpu.VMEM((SB_ROWS, N_ITEMS), jnp.float32),
            pltpu.SemaphoreType.DMA,
        ],
        compiler_params=pltpu.CompilerParams(needs_layout_passes=False),
    )
    def _densify_sc(rows_hbm, cols_hbm, vals_hbm, starts_hbm, x_hbm,
                    rows_v, cols_v, vals_v, starts_v, xbuf, sem):
        wid = lax.axis_index("s") * NC + lax.axis_index("c")

        pltpu.sync_copy(starts_hbm, starts_v)

        zv = jnp.zeros((LANES,), jnp.float32)

        def _memset_row(r):
            def _body(i, carry):
                for u in range(8):
                    xbuf[r, pl.ds((i * 8 + u) * LANES, LANES)] = zv
                return carry
            lax.fori_loop(0, N_ITEMS // (8 * LANES), _body, 0)

        for r in range(SB_ROWS):
            _memset_row(r)

        for t in range(H_SB_PER_W):
            lbase = (wid * H_SB_PER_W + t) * SB_ROWS
            sb = row0 // SB_ROWS + wid * H_SB_PER_W + t
            base = row0 + lbase
            svec = starts_v[pl.ds(sb, LANES)]
            s0 = svec[0]
            s1 = svec[1]
            c0 = (s0 // 8) * 8
            nch = (s1 - c0 + CHUNK - 1) // CHUNK

            def _stage(off):
                d1 = pltpu.async_copy(
                    rows_hbm.at[pl.ds(off, CHUNK)], rows_v, sem)
                d2 = pltpu.async_copy(
                    cols_hbm.at[pl.ds(off, CHUNK)], cols_v, sem)
                d3 = pltpu.async_copy(
                    vals_hbm.at[pl.ds(off, CHUNK)], vals_v, sem)
                d1.wait()
                d2.wait()
                d3.wait()

            def _masked_idx(i):
                rv = rows_v[pl.ds(i * LANES, LANES)]
                cv = cols_v[pl.ds(i * LANES, LANES)]
                m = (rv >= base) & (rv < base + SB_ROWS)
                ri = jnp.where(m, rv - base, 0)
                ci = jnp.where(m, cv, 0)
                return m, ri, ci

            def _chunk_add(j, carry):
                _stage(c0 + j * CHUNK)

                def _vec(i, inner):
                    m, ri, ci = _masked_idx(i)
                    vv = vals_v[pl.ds(i * LANES, LANES)]
                    plsc.addupdate_scatter(
                        xbuf, [ri, ci], jnp.where(m, vv, 0.0), mask=m)
                    return inner

                lax.fori_loop(0, CHUNK // LANES, _vec, 0)
                return carry

            lax.fori_loop(0, nch, _chunk_add, 0)


            pltpu.sync_copy(xbuf, x_hbm.at[pl.ds(lbase, SB_ROWS)])

            def _chunk_zero(j, carry):
                @pl.when(nch > 1)
                def _():
                    _stage(c0 + j * CHUNK)

                def _vec(i, inner):
                    m, ri, ci = _masked_idx(i)
                    plsc.store_scatter(xbuf, [ri, ci], zv, mask=m)
                    return inner

                lax.fori_loop(0, CHUNK // LANES, _vec, 0)
                return carry

            lax.fori_loop(0, nch, _chunk_zero, 0)

    return _densify_sc


_densify_halves = tuple(_make_densify(h * H_ROWS) for h in range(SPLIT))


_BM = 256


def _matmul_first(x, w16):
    k = x.shape[1]
    n = w16.shape[1]

    def body(x_ref, w_ref, o_ref):
        o_ref[...] = jnp.dot(
            x_ref[...].astype(jnp.bfloat16), w_ref[...],
            preferred_element_type=jnp.float32)

    return pl.pallas_call(
        body,
        grid=(H_ROWS // _BM,),
        in_specs=[
            pl.BlockSpec((_BM, k), lambda i: (i, 0)),
            pl.BlockSpec((k, n), lambda i: (0, 0)),
        ],
        out_specs=pl.BlockSpec((_BM, n), lambda i: (i, 0)),
        out_shape=jax.ShapeDtypeStruct((B_USERS, N_ITEMS), jnp.float32),
        compiler_params=pltpu.CompilerParams(
            dimension_semantics=("arbitrary",),
            vmem_limit_bytes=100 * 1024 * 1024),
    )(x, w16)


def _matmul_rest(ybuf, x, w16, h):
    k = x.shape[1]
    n = w16.shape[1]
    nblk = H_ROWS // _BM

    def body(y_ref, x_ref, w_ref, o_ref):
        del y_ref
        o_ref[...] = jnp.dot(
            x_ref[...].astype(jnp.bfloat16), w_ref[...],
            preferred_element_type=jnp.float32)

    return pl.pallas_call(
        body,
        grid=(nblk,),
        in_specs=[
            pl.BlockSpec(memory_space=pltpu.MemorySpace.HBM),
            pl.BlockSpec((_BM, k), lambda i: (i, 0)),
            pl.BlockSpec((k, n), lambda i: (0, 0)),
        ],
        out_specs=pl.BlockSpec((_BM, n), lambda i, _h=h: (i + _h * nblk, 0)),
        out_shape=jax.ShapeDtypeStruct((B_USERS, N_ITEMS), jnp.float32),
        input_output_aliases={0: 0},
        compiler_params=pltpu.CompilerParams(
            dimension_semantics=("arbitrary",),
            vmem_limit_bytes=100 * 1024 * 1024),
    )(ybuf, x, w16)


def kernel(vals, W, rows, cols):
    rows32 = rows.astype(jnp.int32)
    cols32 = cols.astype(jnp.int32)
    vals32 = vals.astype(jnp.float32)
    nnz = rows32.shape[0]

    bounds = jnp.arange(NSB + 1, dtype=jnp.int32) * SB_ROWS
    starts = jnp.sum(rows32[None, :] < bounds[:, None], axis=1,
                     dtype=jnp.int32)
    starts_p = jnp.concatenate(
        [starts, jnp.full((STARTS_PAD - NSB - 1,), nnz, jnp.int32)])

    rows_p = jnp.concatenate(
        [rows32, jnp.full((CHUNK,), B_USERS, jnp.int32)])
    cols_p = jnp.concatenate([cols32, jnp.zeros((CHUNK,), jnp.int32)])
    vals_p = jnp.concatenate([vals32, jnp.zeros((CHUNK,), jnp.float32)])

    w16 = W.astype(jnp.bfloat16)
    xs = [dh(rows_p, cols_p, vals_p, starts_p) for dh in _densify_halves]
    y = _matmul_first(xs[0], w16)
    for h in range(1, SPLIT):
        y = _matmul_rest(y, xs[h], w16, h)
    return y


# speedup vs baseline: 2.4384x; 1.0123x over previous
import functools

import jax
import jax.numpy as jnp
from jax import lax
from jax.experimental import pallas as pl
from jax.experimental.pallas import tpu as pltpu
from jax.experimental.pallas import tpu_sc as plsc

B_USERS = 4096
N_ITEMS = 4096

NC = 2
NS = 16
NW = NC * NS
LANES = 16

SB_ROWS = 16
NSB = B_USERS // SB_ROWS
CHUNK = 2048
STARTS_PAD = 272

SPLIT = 2
H_ROWS = B_USERS // SPLIT
H_NSB = NSB // SPLIT
H_SB_PER_W = H_NSB // NW

_MESH = plsc.VectorSubcoreMesh(core_axis_name="c", subcore_axis_name="s")


def _make_densify(row0):
    @functools.partial(
        pl.kernel,
        out_type=jax.ShapeDtypeStruct((H_ROWS, N_ITEMS), jnp.float32),
        mesh=_MESH,
        scratch_types=[
            pltpu.VMEM((3, CHUNK), jnp.int32),
            pltpu.VMEM((STARTS_PAD,), jnp.int32),
            pltpu.VMEM((SB_ROWS, N_ITEMS), jnp.float32),
            pltpu.SemaphoreType.DMA,
        ],
        compiler_params=pltpu.CompilerParams(needs_layout_passes=False),
    )
    def _densify_sc(coo_hbm, starts_hbm, x_hbm,
                    coo_v, starts_v, xbuf, sem):
        wid = lax.axis_index("s") * NC + lax.axis_index("c")

        pltpu.sync_copy(starts_hbm, starts_v)

        zv = jnp.zeros((LANES,), jnp.float32)

        def _memset_row(r):
            def _body(i, carry):
                for u in range(8):
                    xbuf[r, pl.ds((i * 8 + u) * LANES, LANES)] = zv
                return carry
            lax.fori_loop(0, N_ITEMS // (8 * LANES), _body, 0)

        for r in range(SB_ROWS):
            _memset_row(r)

        for t in range(H_SB_PER_W):
            lbase = (wid * H_SB_PER_W + t) * SB_ROWS
            sb = row0 // SB_ROWS + wid * H_SB_PER_W + t
            base = row0 + lbase
            svec = starts_v[pl.ds(sb, LANES)]
            s0 = svec[0]
            s1 = svec[1]
            c0 = (s0 // 128) * 128
            nch = (s1 - c0 + CHUNK - 1) // CHUNK

            def _stage(off):
                pltpu.async_copy(
                    coo_hbm.at[:, pl.ds(off, CHUNK)], coo_v, sem).wait()

            def _masked_idx(i):
                rv = coo_v[0, pl.ds(i * LANES, LANES)]
                cv = coo_v[1, pl.ds(i * LANES, LANES)]
                m = (rv >= base) & (rv < base + SB_ROWS)
                ri = jnp.where(m, rv - base, 0)
                ci = jnp.where(m, cv, 0)
                return m, ri, ci

            def _nvec(j):
                hi = jnp.minimum(s1, c0 + (j + 1) * CHUNK)
                return jnp.maximum(
                    0, (hi - (c0 + j * CHUNK) + LANES - 1) // LANES)

            def _chunk_add(j, carry):
                _stage(c0 + j * CHUNK)

                def _vec(i, inner):
                    m, ri, ci = _masked_idx(i)
                    vv = plsc.bitcast(
                        coo_v[2, pl.ds(i * LANES, LANES)], jnp.float32)
                    plsc.addupdate_scatter(
                        xbuf, [ri, ci], jnp.where(m, vv, 0.0), mask=m)
                    return inner

                lax.fori_loop(0, _nvec(j), _vec, 0)
                return carry

            lax.fori_loop(0, nch, _chunk_add, 0)


            pltpu.sync_copy(xbuf, x_hbm.at[pl.ds(lbase, SB_ROWS)])

            def _chunk_zero(j, carry):
                @pl.when(nch > 1)
                def _():
                    _stage(c0 + j * CHUNK)

                def _vec(i, inner):
                    m, ri, ci = _masked_idx(i)
                    plsc.store_scatter(xbuf, [ri, ci], zv, mask=m)
                    return inner

                lax.fori_loop(0, _nvec(j), _vec, 0)
                return carry

            lax.fori_loop(0, nch, _chunk_zero, 0)

    return _densify_sc


_densify_halves = tuple(_make_densify(h * H_ROWS) for h in range(SPLIT))


_BM = 256


def _matmul_first(x, w16):
    k = x.shape[1]
    n = w16.shape[1]

    def body(x_ref, w_ref, o_ref):
        o_ref[...] = jnp.dot(
            x_ref[...].astype(jnp.bfloat16), w_ref[...],
            preferred_element_type=jnp.float32)

    return pl.pallas_call(
        body,
        grid=(H_ROWS // _BM,),
        in_specs=[
            pl.BlockSpec((_BM, k), lambda i: (i, 0)),
            pl.BlockSpec((k, n), lambda i: (0, 0)),
        ],
        out_specs=pl.BlockSpec((_BM, n), lambda i: (i, 0)),
        out_shape=jax.ShapeDtypeStruct((B_USERS, N_ITEMS), jnp.float32),
        compiler_params=pltpu.CompilerParams(
            dimension_semantics=("arbitrary",),
            vmem_limit_bytes=100 * 1024 * 1024),
    )(x, w16)


def _matmul_rest(ybuf, x, w16, h):
    k = x.shape[1]
    n = w16.shape[1]
    nblk = H_ROWS // _BM

    def body(y_ref, x_ref, w_ref, o_ref):
        del y_ref
        o_ref[...] = jnp.dot(
            x_ref[...].astype(jnp.bfloat16), w_ref[...],
            preferred_element_type=jnp.float32)

    return pl.pallas_call(
        body,
        grid=(nblk,),
        in_specs=[
            pl.BlockSpec(memory_space=pltpu.MemorySpace.HBM),
            pl.BlockSpec((_BM, k), lambda i: (i, 0)),
            pl.BlockSpec((k, n), lambda i: (0, 0)),
        ],
        out_specs=pl.BlockSpec((_BM, n), lambda i, _h=h: (i + _h * nblk, 0)),
        out_shape=jax.ShapeDtypeStruct((B_USERS, N_ITEMS), jnp.float32),
        input_output_aliases={0: 0},
        compiler_params=pltpu.CompilerParams(
            dimension_semantics=("arbitrary",),
            vmem_limit_bytes=100 * 1024 * 1024),
    )(ybuf, x, w16)


def kernel(vals, W, rows, cols):
    rows32 = rows.astype(jnp.int32)
    cols32 = cols.astype(jnp.int32)
    vals32 = vals.astype(jnp.float32)
    nnz = rows32.shape[0]

    bounds = jnp.arange(NSB + 1, dtype=jnp.int32) * SB_ROWS
    starts = jnp.sum(rows32[None, :] < bounds[:, None], axis=1,
                     dtype=jnp.int32)
    starts_p = jnp.concatenate(
        [starts, jnp.full((STARTS_PAD - NSB - 1,), nnz, jnp.int32)])

    rows_p = jnp.concatenate(
        [rows32, jnp.full((CHUNK,), B_USERS, jnp.int32)])
    cols_p = jnp.concatenate([cols32, jnp.zeros((CHUNK,), jnp.int32)])
    vals_p = jnp.concatenate(
        [lax.bitcast_convert_type(vals32, jnp.int32),
         jnp.zeros((CHUNK,), jnp.int32)])
    coo = jnp.stack([rows_p, cols_p, vals_p])

    w16 = W.astype(jnp.bfloat16)
    xs = [dh(coo, starts_p) for dh in _densify_halves]
    y = _matmul_first(xs[0], w16)
    for h in range(1, SPLIT):
        y = _matmul_rest(y, xs[h], w16, h)
    return y
